# baseline (speedup 1.0000x reference)
import functools

import jax
import jax.numpy as jnp
from jax import lax
from jax.experimental import pallas as pl
from jax.experimental.pallas import tpu as pltpu
from jax.experimental.pallas import tpu_sc as plsc

N = 300
E = 9600
HID = 64
AH = 32
F32 = jnp.float32
EPS = 1e-5

RP = 304
EW0 = 384
EW1 = 256
FLAT = 91648
ZCH = FLAT // 16
RPW = N // 15


def _adj_body(edge, out0, out1, src_v, dst_v, fidx_v, ones_v, zero_v, row_v,
              shared, sem):
    c = lax.axis_index("c")
    s = lax.axis_index("s")

    for j in range(ZCH // 16):
        zero_v[pl.ds(16 * j, 16)] = jnp.zeros((16,), F32)
    pltpu.sync_copy(zero_v, shared.at[pl.ds(s * ZCH, ZCH)])
    for j in range(8):
        ones_v[pl.ds(16 * j, 16)] = jnp.ones((16,), F32)

    def load_fidx(ew, base):
        pltpu.sync_copy(edge.at[0, pl.ds(base, ew)], src_v.at[pl.ds(0, ew)])
        pltpu.sync_copy(edge.at[1, pl.ds(base, ew)], dst_v.at[pl.ds(0, ew)])
        for j in range(ew // 16):
            s16 = src_v[pl.ds(16 * j, 16)]
            d16 = dst_v[pl.ds(16 * j, 16)]
            fidx_v[j // 8, pl.ds((j % 8) * 16, 16)] = d16 * RP + s16

    @pl.when(jnp.logical_and(s < 15, c == 0))
    def _l0():
        load_fidx(EW0, s * EW0)

    @pl.when(jnp.logical_and(s < 15, c == 1))
    def _l1():
        load_fidx(EW1, 15 * EW0 + s * EW1)

    plsc.subcore_barrier()

    @pl.when(jnp.logical_and(s < 15, c == 0))
    def _s0():
        for ch in range(EW0 // 128):
            pltpu.sync_copy(ones_v, shared.at[fidx_v.at[ch]], add=True)

    @pl.when(jnp.logical_and(s < 15, c == 1))
    def _s1():
        for ch in range(EW1 // 128):
            pltpu.sync_copy(ones_v, shared.at[fidx_v.at[ch]], add=True)

    plsc.subcore_barrier()

    @pl.when(jnp.logical_and(s < 15, c == 0))
    def _w0():
        r0 = s * RPW
        for i in range(RPW):
            pltpu.sync_copy(shared.at[pl.ds((r0 + i) * RP, RP)], row_v)
            pltpu.sync_copy(row_v, out0.at[r0 + i])

    @pl.when(jnp.logical_and(s < 15, c == 1))
    def _w1():
        r0 = s * RPW
        for i in range(RPW):
            pltpu.sync_copy(shared.at[pl.ds((r0 + i) * RP, RP)], row_v)
            pltpu.sync_copy(row_v, out1.at[r0 + i])


def _adj_sc(edge_index):
    mesh = plsc.VectorSubcoreMesh(core_axis_name="c", subcore_axis_name="s")
    kern = pl.kernel(
        _adj_body,
        out_type=(jax.ShapeDtypeStruct((N, RP), F32),
                  jax.ShapeDtypeStruct((N, RP), F32)),
        mesh=mesh,
        scratch_types=[
            pltpu.VMEM((EW0,), jnp.int32),
            pltpu.VMEM((EW0,), jnp.int32),
            pltpu.VMEM((EW0 // 128, 128), jnp.int32),
            pltpu.VMEM((128,), F32),
            pltpu.VMEM((ZCH,), F32),
            pltpu.VMEM((RP,), F32),
            pltpu.VMEM_SHARED((FLAT,), F32),
            pltpu.SemaphoreType.DMA,
        ],
    )
    return kern(edge_index)


def _fwd_body(adj0, adj1, feat,
              W01, b01, ga1, be1, W11, b11,
              W02, b02, ga2, be2, W12, b12,
              Wa0, ba0, Wa1,
              Wc0, bc0, Wc1, bc1,
              pi_ref, val_ref):
    adjv = adj0[...] + adj1[...]

    def gin(x, W0, b0, ga, be, W1, b1):
        xp = jnp.concatenate(
            [x, jnp.zeros((RP - N, x.shape[1]), F32)], axis=0)
        xa = x + jnp.dot(adjv, xp, preferred_element_type=F32)
        h = jnp.dot(xa, W0[...], preferred_element_type=F32) + b0[...]
        mu = jnp.mean(h, axis=0, keepdims=True)
        var = jnp.mean((h - mu) ** 2, axis=0, keepdims=True)
        h = ga[...] * (h - mu) / jnp.sqrt(var + EPS) + be[...]
        h = jnp.maximum(h, 0.0)
        return jnp.dot(h, W1[...], preferred_element_type=F32) + b1[...]

    x1 = gin(feat[...], W01, b01, ga1, be1, W11, b11)
    x2 = gin(x1, W02, b02, ga2, be2, W12, b12)

    g = jnp.mean(x2, axis=0, keepdims=True)

    hc = jnp.maximum(jnp.dot(g, Wc0[...], preferred_element_type=F32)
                     + bc0[...], 0.0)
    val_ref[...] = jnp.dot(hc, Wc1[...], preferred_element_type=F32) + bc1[...]

    AT = lax.dot_general(Wa0[HID:2 * HID, :], x2, (((0,), (1,)), ((), ())),
                         preferred_element_type=F32)
    B = jnp.dot(x2, Wa0[2 * HID:, :], preferred_element_type=F32)
    gA = jnp.dot(g, Wa0[:HID, :], preferred_element_type=F32) \
        + ba0[...][None, :]

    L = jnp.zeros((N, N), F32)
    for k in range(AH):
        zk = AT[k:k + 1, :] + B[:, k:k + 1] + gA[0:1, k:k + 1]
        L = L + jnp.maximum(zk, 0.0) * Wa1[k:k + 1, 0:1]
    m = jnp.max(L, keepdims=True)
    ex = jnp.exp(L - m)
    pi_ref[...] = ex / jnp.sum(ex, keepdims=True)


_OUT_SHAPE = (jax.ShapeDtypeStruct((N, N), F32),
              jax.ShapeDtypeStruct((1, 1), F32))


def _flat_args(adj0, adj1, features, params):
    gp = params['gin']
    ap = params['actor']
    cp = params['critic']
    return [
        adj0, adj1, features,
        gp[0]['W0'], gp[0]['b0'], gp[0]['gamma'], gp[0]['beta'],
        gp[0]['W1'], gp[0]['b1'],
        gp[1]['W0'], gp[1]['b0'], gp[1]['gamma'], gp[1]['beta'],
        gp[1]['W1'], gp[1]['b1'],
        ap['W0'], ap['b0'], ap['W1'],
        cp['W0'], cp['b0'], cp['W1'], cp['b1'].reshape(1, 1),
    ]


def kernel(features, edge_index, params):
    adj0, adj1 = _adj_sc(edge_index)
    args = _flat_args(adj0, adj1, features, params)
    pi300, val = pl.pallas_call(_fwd_body, out_shape=_OUT_SHAPE)(*args)
    return (pi300.reshape(N * N, 1), val)


# speedup vs baseline: 1.0185x; 1.0185x over previous
import functools

import jax
import jax.numpy as jnp
from jax import lax
from jax.experimental import pallas as pl
from jax.experimental.pallas import tpu as pltpu
from jax.experimental.pallas import tpu_sc as plsc

N = 300
E = 9600
HID = 64
AH = 32
F32 = jnp.float32
EPS = 1e-5

RP = 304
EW0 = 384
EW1 = 256
FLAT = 91648
ZCH = FLAT // 16
RPW = N // 15


def _adj_body(edge, out0, out1, src_v, dst_v, fidx_v, ones_v, zero_v, row_v,
              shared, sem):
    c = lax.axis_index("c")
    s = lax.axis_index("s")

    for j in range(ZCH // 16):
        zero_v[pl.ds(16 * j, 16)] = jnp.zeros((16,), F32)
    pltpu.sync_copy(zero_v, shared.at[pl.ds(s * ZCH, ZCH)])
    for j in range(8):
        ones_v[pl.ds(16 * j, 16)] = jnp.ones((16,), F32)

    def load_fidx(ew, base):
        pltpu.sync_copy(edge.at[0, pl.ds(base, ew)], src_v.at[pl.ds(0, ew)])
        pltpu.sync_copy(edge.at[1, pl.ds(base, ew)], dst_v.at[pl.ds(0, ew)])
        for j in range(ew // 16):
            s16 = src_v[pl.ds(16 * j, 16)]
            d16 = dst_v[pl.ds(16 * j, 16)]
            fidx_v[j // 8, pl.ds((j % 8) * 16, 16)] = d16 * RP + s16

    @pl.when(jnp.logical_and(s < 15, c == 0))
    def _l0():
        load_fidx(EW0, s * EW0)

    @pl.when(jnp.logical_and(s < 15, c == 1))
    def _l1():
        load_fidx(EW1, 15 * EW0 + s * EW1)

    plsc.subcore_barrier()

    @pl.when(jnp.logical_and(s < 15, c == 0))
    def _s0():
        for ch in range(EW0 // 128):
            pltpu.sync_copy(ones_v, shared.at[fidx_v.at[ch]], add=True)

    @pl.when(jnp.logical_and(s < 15, c == 1))
    def _s1():
        for ch in range(EW1 // 128):
            pltpu.sync_copy(ones_v, shared.at[fidx_v.at[ch]], add=True)

    plsc.subcore_barrier()

    def writeout(out):
        r0 = s * RPW
        pltpu.sync_copy(shared.at[pl.ds(r0 * RP, RPW * RP)], row_v)
        pltpu.sync_copy(row_v, out.at[pl.ds(r0 * RP, RPW * RP)])

    @pl.when(jnp.logical_and(s < 15, c == 0))
    def _w0():
        writeout(out0)

    @pl.when(jnp.logical_and(s < 15, c == 1))
    def _w1():
        writeout(out1)


def _adj_sc(edge_index):
    mesh = plsc.VectorSubcoreMesh(core_axis_name="c", subcore_axis_name="s")
    kern = pl.kernel(
        _adj_body,
        out_type=(jax.ShapeDtypeStruct((N * RP,), F32),
                  jax.ShapeDtypeStruct((N * RP,), F32)),
        mesh=mesh,
        scratch_types=[
            pltpu.VMEM((EW0,), jnp.int32),
            pltpu.VMEM((EW0,), jnp.int32),
            pltpu.VMEM((EW0 // 128, 128), jnp.int32),
            pltpu.VMEM((128,), F32),
            pltpu.VMEM((ZCH,), F32),
            pltpu.VMEM((RPW * RP,), F32),
            pltpu.VMEM_SHARED((FLAT,), F32),
            pltpu.SemaphoreType.DMA,
        ],
    )
    return kern(edge_index)


def _fwd_body(adj0, adj1, feat,
              W01, b01, ga1, be1, W11, b11,
              W02, b02, ga2, be2, W12, b12,
              Wa0, ba0, Wa1,
              Wc0, bc0, Wc1, bc1,
              pi_ref, val_ref):
    adjv = adj0[...] + adj1[...]

    def gin(x, W0, b0, ga, be, W1, b1):
        xp = jnp.concatenate(
            [x, jnp.zeros((RP - N, x.shape[1]), F32)], axis=0)
        xa = x + jnp.dot(adjv, xp, preferred_element_type=F32)
        h = jnp.dot(xa, W0[...], preferred_element_type=F32) + b0[...]
        mu = jnp.mean(h, axis=0, keepdims=True)
        var = jnp.mean((h - mu) ** 2, axis=0, keepdims=True)
        h = ga[...] * (h - mu) / jnp.sqrt(var + EPS) + be[...]
        h = jnp.maximum(h, 0.0)
        return jnp.dot(h, W1[...], preferred_element_type=F32) + b1[...]

    x1 = gin(feat[...], W01, b01, ga1, be1, W11, b11)
    x2 = gin(x1, W02, b02, ga2, be2, W12, b12)

    g = jnp.mean(x2, axis=0, keepdims=True)

    hc = jnp.maximum(jnp.dot(g, Wc0[...], preferred_element_type=F32)
                     + bc0[...], 0.0)
    val_ref[...] = jnp.dot(hc, Wc1[...], preferred_element_type=F32) + bc1[...]

    AT = lax.dot_general(Wa0[HID:2 * HID, :], x2, (((0,), (1,)), ((), ())),
                         preferred_element_type=F32)
    B = jnp.dot(x2, Wa0[2 * HID:, :], preferred_element_type=F32)
    gA = jnp.dot(g, Wa0[:HID, :], preferred_element_type=F32) \
        + ba0[...][None, :]

    L = jnp.zeros((N, N), F32)
    for k in range(AH):
        zk = AT[k:k + 1, :] + B[:, k:k + 1] + gA[0:1, k:k + 1]
        L = L + jnp.maximum(zk, 0.0) * Wa1[k:k + 1, 0:1]
    m = jnp.max(L, keepdims=True)
    ex = jnp.exp(L - m)
    pi_ref[...] = ex / jnp.sum(ex, keepdims=True)


_OUT_SHAPE = (jax.ShapeDtypeStruct((N, N), F32),
              jax.ShapeDtypeStruct((1, 1), F32))


def _flat_args(adj0, adj1, features, params):
    gp = params['gin']
    ap = params['actor']
    cp = params['critic']
    return [
        adj0, adj1, features,
        gp[0]['W0'], gp[0]['b0'], gp[0]['gamma'], gp[0]['beta'],
        gp[0]['W1'], gp[0]['b1'],
        gp[1]['W0'], gp[1]['b0'], gp[1]['gamma'], gp[1]['beta'],
        gp[1]['W1'], gp[1]['b1'],
        ap['W0'], ap['b0'], ap['W1'],
        cp['W0'], cp['b0'], cp['W1'], cp['b1'].reshape(1, 1),
    ]


def kernel(features, edge_index, params):
    adj0f, adj1f = _adj_sc(edge_index)
    adj0 = adj0f.reshape(N, RP)
    adj1 = adj1f.reshape(N, RP)
    args = _flat_args(adj0, adj1, features, params)
    pi300, val = pl.pallas_call(_fwd_body, out_shape=_OUT_SHAPE)(*args)
    return (pi300.reshape(N * N, 1), val)


# speedup vs baseline: 1.0534x; 1.0342x over previous
import functools

import jax
import jax.numpy as jnp
from jax import lax
from jax.experimental import pallas as pl
from jax.experimental.pallas import tpu as pltpu
from jax.experimental.pallas import tpu_sc as plsc

N = 300
E = 9600
HID = 64
AH = 32
F32 = jnp.float32
EPS = 1e-5

RP = 304
EW0 = 384
EW1 = 256
FLAT = 91648
ZCH = FLAT // 16
RPW = N // 15


def _adj_body(edge, out0, out1, src_v, dst_v, fidx_v, ones_v, zero_v, row_v,
              shared, sem):
    c = lax.axis_index("c")
    s = lax.axis_index("s")

    for j in range(ZCH // 16):
        zero_v[pl.ds(16 * j, 16)] = jnp.zeros((16,), F32)
    pltpu.sync_copy(zero_v, shared.at[pl.ds(s * ZCH, ZCH)])
    for j in range(8):
        ones_v[pl.ds(16 * j, 16)] = jnp.ones((16,), F32)

    def load_fidx(ew, base):
        pltpu.sync_copy(edge.at[0, pl.ds(base, ew)], src_v.at[pl.ds(0, ew)])
        pltpu.sync_copy(edge.at[1, pl.ds(base, ew)], dst_v.at[pl.ds(0, ew)])
        for j in range(ew // 16):
            s16 = src_v[pl.ds(16 * j, 16)]
            d16 = dst_v[pl.ds(16 * j, 16)]
            fidx_v[j // 8, pl.ds((j % 8) * 16, 16)] = d16 * RP + s16

    @pl.when(jnp.logical_and(s < 15, c == 0))
    def _l0():
        load_fidx(EW0, s * EW0)

    @pl.when(jnp.logical_and(s < 15, c == 1))
    def _l1():
        load_fidx(EW1, 15 * EW0 + s * EW1)

    plsc.subcore_barrier()

    @pl.when(jnp.logical_and(s < 15, c == 0))
    def _s0():
        for ch in range(EW0 // 128):
            pltpu.sync_copy(ones_v, shared.at[fidx_v.at[ch]], add=True)

    @pl.when(jnp.logical_and(s < 15, c == 1))
    def _s1():
        for ch in range(EW1 // 128):
            pltpu.sync_copy(ones_v, shared.at[fidx_v.at[ch]], add=True)

    plsc.subcore_barrier()

    def writeout(out):
        r0 = s * RPW
        pltpu.sync_copy(shared.at[pl.ds(r0 * RP, RPW * RP)], row_v)
        pltpu.sync_copy(row_v, out.at[pl.ds(r0 * RP, RPW * RP)])

    @pl.when(jnp.logical_and(s < 15, c == 0))
    def _w0():
        writeout(out0)

    @pl.when(jnp.logical_and(s < 15, c == 1))
    def _w1():
        writeout(out1)


def _adj_sc(edge_index):
    mesh = plsc.VectorSubcoreMesh(core_axis_name="c", subcore_axis_name="s")
    kern = pl.kernel(
        _adj_body,
        out_type=(jax.ShapeDtypeStruct((N * RP,), F32),
                  jax.ShapeDtypeStruct((N * RP,), F32)),
        mesh=mesh,
        scratch_types=[
            pltpu.VMEM((EW0,), jnp.int32),
            pltpu.VMEM((EW0,), jnp.int32),
            pltpu.VMEM((EW0 // 128, 128), jnp.int32),
            pltpu.VMEM((128,), F32),
            pltpu.VMEM((ZCH,), F32),
            pltpu.VMEM((RPW * RP,), F32),
            pltpu.VMEM_SHARED((FLAT,), F32),
            pltpu.SemaphoreType.DMA,
        ],
    )
    return kern(edge_index)


def _fwd_body(adj, feat,
              W01, b01, ga1, be1, W11, b11,
              W02, b02, ga2, be2, W12, b12,
              Wa0, ba0, Wa1,
              Wc0, bc0, Wc1, bc1,
              pi_ref, val_ref):
    adjv = adj[...]

    def gin(x, W0, b0, ga, be, W1, b1):
        xp = jnp.concatenate(
            [x, jnp.zeros((RP - N, x.shape[1]), F32)], axis=0)
        xa = x + jnp.dot(adjv, xp, preferred_element_type=F32)
        h = jnp.dot(xa, W0[...], preferred_element_type=F32) + b0[...]
        mu = jnp.mean(h, axis=0, keepdims=True)
        var = jnp.mean((h - mu) ** 2, axis=0, keepdims=True)
        h = ga[...] * (h - mu) / jnp.sqrt(var + EPS) + be[...]
        h = jnp.maximum(h, 0.0)
        return jnp.dot(h, W1[...], preferred_element_type=F32) + b1[...]

    x1 = gin(feat[...], W01, b01, ga1, be1, W11, b11)
    x2 = gin(x1, W02, b02, ga2, be2, W12, b12)

    g = jnp.mean(x2, axis=0, keepdims=True)

    hc = jnp.maximum(jnp.dot(g, Wc0[...], preferred_element_type=F32)
                     + bc0[...], 0.0)
    val_ref[...] = jnp.dot(hc, Wc1[...], preferred_element_type=F32) + bc1[...]

    AT = lax.dot_general(Wa0[HID:2 * HID, :], x2, (((0,), (1,)), ((), ())),
                         preferred_element_type=F32)
    B = jnp.dot(x2, Wa0[2 * HID:, :], preferred_element_type=F32)
    gA = jnp.dot(g, Wa0[:HID, :], preferred_element_type=F32) \
        + ba0[...][None, :]

    L = jnp.zeros((N, N), F32)
    for k in range(AH):
        zk = AT[k:k + 1, :] + B[:, k:k + 1] + gA[0:1, k:k + 1]
        L = L + jnp.maximum(zk, 0.0) * Wa1[k:k + 1, 0:1]
    m = jnp.max(L, keepdims=True)
    ex = jnp.exp(L - m)
    pi_ref[...] = ex / jnp.sum(ex, keepdims=True)


_OUT_SHAPE = (jax.ShapeDtypeStruct((N, N), F32),
              jax.ShapeDtypeStruct((1, 1), F32))


def _flat_args(adj, features, params):
    gp = params['gin']
    ap = params['actor']
    cp = params['critic']
    return [
        adj, features,
        gp[0]['W0'], gp[0]['b0'], gp[0]['gamma'], gp[0]['beta'],
        gp[0]['W1'], gp[0]['b1'],
        gp[1]['W0'], gp[1]['b0'], gp[1]['gamma'], gp[1]['beta'],
        gp[1]['W1'], gp[1]['b1'],
        ap['W0'], ap['b0'], ap['W1'],
        cp['W0'], cp['b0'], cp['W1'], cp['b1'].reshape(1, 1),
    ]


def kernel(features, edge_index, params):
    adj0f, adj1f = _adj_sc(edge_index)
    adj = (adj0f + adj1f).reshape(N, RP)
    args = _flat_args(adj, features, params)
    pi300, val = pl.pallas_call(_fwd_body, out_shape=_OUT_SHAPE)(*args)
    return (pi300.reshape(N * N, 1), val)
